# baseline (speedup 1.0000x reference)
import jax
import jax.numpy as jnp
from jax.experimental import pallas as pl

N = 8192
D = 128


def _graph_layer_body(nodes_ref, w2_ref, b2_ref, w3_ref, b3_ref,
                      w5_ref, b5_ref, out_ref):
    nodes = nodes_ref[:]
    w25 = jax.lax.dot_general(
        w5_ref[:], w2_ref[:], (((1,), (0,)), ((), ())),
        preferred_element_type=jnp.float32)
    c25 = jax.lax.dot_general(
        b2_ref[:], w5_ref[:], (((1,), (1,)), ((), ())),
        preferred_element_type=jnp.float32)
    w_cat = jnp.concatenate([w3_ref[:], w25], axis=0)
    b_cat = jnp.concatenate([b3_ref[:], c25], axis=1)
    hg = jax.lax.dot_general(
        nodes, w_cat, (((1,), (1,)), ((), ())),
        preferred_element_type=jnp.float32) + b_cat
    h = hg[:, :D]
    g = hg[:, D:]
    t = jax.lax.dot_general(
        h, g, (((0,), (0,)), ((), ())),
        preferred_element_type=jnp.float32)
    d = jnp.sum(h * h, axis=1, keepdims=True)
    inv = 1.0 / (N - 1)
    out_ref[:] = (jax.lax.dot_general(
        h, t, (((1,), (0,)), ((), ())),
        preferred_element_type=jnp.float32) - d * g) * inv + b5_ref[:] + nodes


@jax.jit
def kernel(nodes_in, inputs, W2, b2, W3, b3, W5, b5):
    del inputs
    b2r = b2.reshape(1, D)
    b3r = b3.reshape(1, D)
    b5r = b5.reshape(1, D)
    return pl.pallas_call(
        _graph_layer_body,
        out_shape=jax.ShapeDtypeStruct((N, D), jnp.float32),
    )(nodes_in, W2, b2r, W3, b3r, W5, b5r)
